# baseline (speedup 1.0000x reference)
import functools

import jax
import jax.numpy as jnp
from jax import lax
from jax.experimental import pallas as pl
from jax.experimental.pallas import tpu as pltpu
from jax.experimental.pallas import tpu_sc as plsc

_NUM = 16
_NC = 2
_NS = 16


def kernel(x):
    b, c, t, h, w = x.shape
    bc = b * c
    d = h * w
    rows_out = bc * _NUM
    nw = _NC * _NS
    per = rows_out // nw

    xr = x.reshape(bc * t, d)
    mesh = plsc.VectorSubcoreMesh(
        core_axis_name="c", subcore_axis_name="s",
        num_cores=_NC, num_subcores=_NS,
    )

    @functools.partial(
        pl.kernel,
        out_type=jax.ShapeDtypeStruct((rows_out, d), x.dtype),
        mesh=mesh,
        scratch_types=[pltpu.SemaphoreType.DMA],
    )
    def sc_gather(x_hbm, out_hbm, sem):
        wid = lax.axis_index("s") * _NC + lax.axis_index("c")
        base = wid * per
        copies = []
        for k in range(per):
            r = base + k
            g = r // _NUM
            j = r - g * _NUM
            src = g * t + (j * (t - 1)) // (_NUM - 1)
            cp = pltpu.make_async_copy(x_hbm.at[src], out_hbm.at[r], sem)
            cp.start()
            copies.append(cp)
        for cp in copies:
            cp.wait()

    out = sc_gather(xr)
    return out.reshape(b, c, _NUM, h, w)


# speedup vs baseline: 5.6233x; 5.6233x over previous
import functools

import jax
import jax.numpy as jnp
from jax import lax
from jax.experimental import pallas as pl
from jax.experimental.pallas import tpu as pltpu
from jax.experimental.pallas import tpu_sc as plsc

_NUM = 16
_NC = 2
_NS = 16


def kernel(x):
    b, c, t, h, w = x.shape
    bc = b * c
    d = h * w
    rows_out = bc * _NUM
    nw = _NC * _NS
    per = rows_out // nw

    xr = x.reshape(bc * t, d)
    mesh = plsc.VectorSubcoreMesh(
        core_axis_name="c", subcore_axis_name="s",
        num_cores=_NC, num_subcores=_NS,
    )

    @functools.partial(
        pl.kernel,
        out_type=jax.ShapeDtypeStruct((rows_out, d), x.dtype),
        mesh=mesh,
        scratch_types=[
            pltpu.VMEM((2, d), jnp.float32),
            pltpu.SemaphoreType.DMA,
            pltpu.SemaphoreType.DMA,
            pltpu.SemaphoreType.DMA,
        ],
    )
    def sc_gather(x_hbm, out_hbm, buf, sem_in, sem_out0, sem_out1):
        wid = lax.axis_index("s") * _NC + lax.axis_index("c")
        base = wid * per
        sems_out = (sem_out0, sem_out1)

        def fetch(k):
            r = base + k
            g = r // _NUM
            j = r - g * _NUM
            src = g * t + (j * (t - 1)) // (_NUM - 1)
            return pltpu.make_async_copy(x_hbm.at[src], buf.at[k % 2], sem_in)

        def store(k):
            return pltpu.make_async_copy(
                buf.at[k % 2], out_hbm.at[base + k], sems_out[k % 2])

        stores = [None] * per
        fetch(0).start()
        for k in range(per):
            fetch(k).wait()
            stores[k] = store(k)
            stores[k].start()
            if k + 1 < per:
                if k >= 1:
                    stores[k - 1].wait()
                fetch(k + 1).start()
        stores[per - 1].wait()

    out = sc_gather(xr)
    return out.reshape(b, c, _NUM, h, w)


# speedup vs baseline: 6.7062x; 1.1926x over previous
import functools

import jax
import jax.numpy as jnp
from jax import lax
from jax.experimental import pallas as pl
from jax.experimental.pallas import tpu as pltpu
from jax.experimental.pallas import tpu_sc as plsc

_NUM = 16
_K = 1
_NC = 2
_NS = 16


def _tc_body(idx_ref, in_ref, out_ref):
    out_ref[...] = in_ref[...]


def kernel(x):
    b, c, t, h, w = x.shape
    bc = b * c
    d = h * w
    nw = _NC * _NS

    xr = x.reshape(bc * t, d)
    mesh = plsc.VectorSubcoreMesh(
        core_axis_name="c", subcore_axis_name="s",
        num_cores=_NC, num_subcores=_NS,
    )

    nrows = bc * _K
    per = -(-nrows // nw)

    @functools.partial(
        pl.kernel,
        out_type=jax.ShapeDtypeStruct((nrows, d), x.dtype),
        mesh=mesh,
        scratch_types=[
            pltpu.VMEM((2, d), jnp.float32),
            pltpu.SemaphoreType.DMA,
            pltpu.SemaphoreType.DMA,
            pltpu.SemaphoreType.DMA,
        ],
    )
    def sc_gather(x_hbm, out_hbm, buf, sem_in, sem_out0, sem_out1):
        wid = lax.axis_index("s") * _NC + lax.axis_index("c")
        base = wid * per
        n_mine = jnp.minimum(nrows - base, per)
        sems_out = (sem_out0, sem_out1)

        def fetch(k):
            r = base + k
            g = r // _K
            j = r - g * _K
            src = g * t + (j * (t - 1)) // (_NUM - 1)
            return pltpu.make_async_copy(x_hbm.at[src], buf.at[k % 2], sem_in)

        def store(k):
            return pltpu.make_async_copy(
                buf.at[k % 2], out_hbm.at[base + k], sems_out[k % 2])

        for k in range(per):
            @pl.when(k < n_mine)
            def _():
                fetch(k).start()
                fetch(k).wait()
                store(k).start()
                store(k).wait()

    sc_part = sc_gather(xr)

    idx = jnp.clip(jnp.linspace(0.0, t - 1, _NUM), 0, t - 1).astype(jnp.int32)
    xr4 = x.reshape(bc, t, h, w)
    out_tc = pl.pallas_call(
        _tc_body,
        grid_spec=pltpu.PrefetchScalarGridSpec(
            num_scalar_prefetch=1,
            grid=(_NUM - _K,),
            in_specs=[
                pl.BlockSpec((bc, 1, h, w),
                             lambda j, idx_ref: (0, idx_ref[j + _K], 0, 0)),
            ],
            out_specs=pl.BlockSpec((bc, 1, h, w),
                                   lambda j, idx_ref: (0, j + _K, 0, 0)),
        ),
        out_shape=jax.ShapeDtypeStruct((bc, _NUM, h, w), x.dtype),
    )(idx, xr4)

    out = lax.dynamic_update_slice(
        out_tc, sc_part.reshape(bc, _K, h, w), (0, 0, 0, 0))
    return out.reshape(b, c, _NUM, h, w)


# speedup vs baseline: 33.5119x; 4.9971x over previous
import functools

import jax
import jax.numpy as jnp
from jax import lax
from jax.experimental import pallas as pl
from jax.experimental.pallas import tpu as pltpu
from jax.experimental.pallas import tpu_sc as plsc

_NUM = 16
_NC = 2
_NS = 16


def kernel(x):
    b, c, t, h, w = x.shape
    bc = b * c
    rows_out = bc * _NUM
    nw = _NC * _NS
    per = rows_out // nw

    xr = x.reshape(bc * t, h, w)
    mesh = plsc.VectorSubcoreMesh(
        core_axis_name="c", subcore_axis_name="s",
        num_cores=_NC, num_subcores=_NS,
    )

    @functools.partial(
        pl.kernel,
        out_type=jax.ShapeDtypeStruct((rows_out, h, w), x.dtype),
        mesh=mesh,
        scratch_types=[
            pltpu.VMEM((2, h, w), jnp.float32),
            pltpu.SemaphoreType.DMA,
            pltpu.SemaphoreType.DMA,
            pltpu.SemaphoreType.DMA,
        ],
    )
    def sc_gather(x_hbm, out_hbm, buf, sem_in, sem_out0, sem_out1):
        wid = lax.axis_index("s") * _NC + lax.axis_index("c")
        base = wid * per
        sems_out = (sem_out0, sem_out1)

        def fetch(k):
            r = base + k
            g = r // _NUM
            j = r - g * _NUM
            src = g * t + (j * (t - 1)) // (_NUM - 1)
            return pltpu.make_async_copy(x_hbm.at[src], buf.at[k % 2], sem_in)

        def store(k):
            return pltpu.make_async_copy(
                buf.at[k % 2], out_hbm.at[base + k], sems_out[k % 2])

        stores = [None] * per
        fetch(0).start()
        for k in range(per):
            fetch(k).wait()
            stores[k] = store(k)
            stores[k].start()
            if k + 1 < per:
                if k >= 1:
                    stores[k - 1].wait()
                fetch(k + 1).start()
        stores[per - 1].wait()

    out = sc_gather(xr)
    return out.reshape(b, c, _NUM, h, w)
